# Initial kernel scaffold; baseline (speedup 1.0000x reference)
import jax
import jax.numpy as jnp
from jax.experimental import pallas as pl


def kernel(x, edge_index, time_diffs, W_S, b_S, W_N, b_N):
    raise NotImplementedError("write your pallas kernel here")


# speedup vs baseline: 8.7434x; 8.7434x over previous
import functools

import jax
import jax.numpy as jnp
from jax import lax
from jax.experimental import pallas as pl
from jax.experimental.pallas import tpu as pltpu
from jax.experimental.pallas import tpu_sc as plsc

N_NODES = 10000
D_IN = 128
D_OUT = 128
NC = 2
NS = 16
NW = NC * NS
CH = 128
HALF = CH // 2
NACC = 10112
RPT = NACC // NS
WROWS = 80
NPAD = 10240
BLK = 1024


def _lane_splat(v, l):
    return lax.gather(
        v, jnp.full((16, 1), l, jnp.int32),
        lax.GatherDimensionNumbers(offset_dims=(), collapsed_slice_dims=(0,),
                                   start_index_map=(0,)),
        (1,), mode=lax.GatherScatterMode.PROMISE_IN_BOUNDS)


def _sc_aggregate(pk, x, nchunks):
    mesh = plsc.VectorSubcoreMesh(core_axis_name="c", subcore_axis_name="s")

    @functools.partial(
        pl.kernel,
        mesh=mesh,
        out_type=(
            jax.ShapeDtypeStruct((NC, NPAD, D_IN), jnp.float32),
            jax.ShapeDtypeStruct((NC, NS, WROWS, 128), jnp.float32),
        ),
        scratch_types=[
            pltpu.VMEM((2, 6, HALF), jnp.int32),
            pltpu.VMEM((CH, D_IN), jnp.float32),
            pltpu.VMEM((WROWS, 128), jnp.float32),
            pltpu.VMEM_SHARED((NACC, D_IN), jnp.float32),
            pltpu.SemaphoreType.DMA,
            pltpu.SemaphoreType.DMA,
            pltpu.SemaphoreType.DMA,
            pltpu.SemaphoreType.DMA,
            pltpu.SemaphoreType.DMA,
        ],
    )
    def k(pk_hbm, x_hbm, feat_hbm, wsum_hbm,
          pkv, gv, wloc, acc, sga, sgb, ssa, ssb, si):
        cid = lax.axis_index("c")
        sid = lax.axis_index("s")
        wid = sid * NC + cid
        zero16 = jnp.zeros((16,), jnp.float32)
        lane = lax.iota(jnp.int32, 16)

        def _zw(r, carry):
            for q in range(128 // 16):
                wloc[r, pl.ds(16 * q, 16)] = zero16
            return carry
        lax.fori_loop(0, WROWS, _zw, 0)

        def _zrow(r, carry):
            for q in range(D_IN // 16):
                gv[r, pl.ds(16 * q, 16)] = zero16
            return carry
        lax.fori_loop(0, CH, _zrow, 0)
        base = sid * RPT
        for i in range(4):
            pltpu.sync_copy(gv, acc.at[pl.ds(base + i * CH, CH)])
        pltpu.sync_copy(gv.at[pl.ds(0, RPT - 4 * CH)],
                        acc.at[pl.ds(base + 4 * CH, RPT - 4 * CH)])
        plsc.subcore_barrier()

        n_my = (nchunks - wid + NW - 1) // NW

        def feat_half(par, h):
            def grp(j, carry2):
                tdi = pkv[par, 4 + h, pl.ds(j * 16, 16)]
                w16 = jnp.exp(tdi.astype(jnp.float32) * (-1.0 / 16777216.0))
                for l in range(16):
                    wspl = _lane_splat(w16, l)
                    e = h * HALF + j * 16 + l
                    for q in range(D_IN // 16):
                        gv[e, pl.ds(16 * q, 16)] = gv[e, pl.ds(16 * q, 16)] * wspl
                return carry2
            lax.fori_loop(0, HALF // 16, grp, 0)

        def wsum_chunk(par):
            def grp(j8, carry2):
                h = j8 // (HALF // 16)
                j = j8 % (HALF // 16)
                tdi = pkv[par, 4 + h, pl.ds(j * 16, 16)]
                w16 = jnp.exp(tdi.astype(jnp.float32) * (-1.0 / 16777216.0))
                r16 = pkv[par, h, pl.ds(j * 16, 16)]
                m16 = r16 & 15
                for l in range(16):
                    mspl = _lane_splat(m16, l)
                    hot = jnp.where(lane == mspl, _lane_splat(w16, l), zero16)
                    ue = r16[l]
                    rw = lax.shift_right_logical(ue, 7)
                    c0 = pl.multiple_of((lax.shift_right_logical(ue, 4) & 7) * 16, 16)
                    wloc[rw, pl.ds(c0, 16)] = wloc[rw, pl.ds(c0, 16)] + hot
                return carry2
            lax.fori_loop(0, CH // 16, grp, 0)

        def _drain(sem, rows, nbytes_ref_rows):
            pltpu.make_async_copy(
                feat_hbm.at[0, pl.ds(0, rows)], gv.at[pl.ds(0, rows)], sem
            ).wait()

        pltpu.async_copy(pk_hbm.at[wid], pkv.at[0], si)

        def body(t, carry):
            par = t & 1
            c = wid + t * NW
            pltpu.make_async_copy(pk_hbm.at[0], pkv.at[par], si).wait()

            @pl.when(t > 0)
            def _():
                _drain(ssa, HALF, None)
            ga = pltpu.async_copy(x_hbm.at[pkv.at[par, 2]], gv.at[pl.ds(0, HALF)], sga)

            @pl.when(t > 0)
            def _():
                _drain(ssb, HALF, None)
            gb = pltpu.async_copy(x_hbm.at[pkv.at[par, 3]], gv.at[pl.ds(HALF, HALF)], sgb)

            @pl.when(t > 0)
            def _():
                wsum_chunk(1 - par)

            @pl.when(t + 1 < n_my)
            def _():
                pltpu.async_copy(pk_hbm.at[c + NW], pkv.at[1 - par], si)

            ga.wait()
            feat_half(par, 0)
            pltpu.async_copy(gv.at[pl.ds(0, HALF)], acc.at[pkv.at[par, 0]],
                             ssa, add=True)
            gb.wait()
            feat_half(par, 1)
            pltpu.async_copy(gv.at[pl.ds(HALF, HALF)], acc.at[pkv.at[par, 1]],
                             ssb, add=True)
            return carry
        lax.fori_loop(0, n_my, body, 0)

        wsum_chunk((n_my - 1) & 1)
        _drain(ssa, HALF, None)
        _drain(ssb, HALF, None)
        pltpu.sync_copy(wloc, wsum_hbm.at[cid, sid])
        plsc.subcore_barrier()
        pltpu.sync_copy(acc.at[pl.ds(base, RPT)], feat_hbm.at[cid, pl.ds(base, RPT)])

    return k(pk, x)


def _tc_final(x_pad, feat, wflat, ws_t, wn_t, b_s, b_n):
    grid = (NPAD // BLK,)

    def body(x_ref, pf_ref, wf_ref, ws_ref, wn_ref, bs_ref, bn_ref, o_ref):
        agg = pf_ref[0] + pf_ref[1]
        acc = jnp.dot(x_ref[...], ws_ref[...], preferred_element_type=jnp.float32)
        acc += jnp.dot(agg, wn_ref[...], preferred_element_type=jnp.float32)
        bn_b = jnp.broadcast_to(bn_ref[...], (NW, D_OUT))
        acc += lax.dot_general(wf_ref[...], bn_b, (((0,), (0,)), ((), ())),
                               preferred_element_type=jnp.float32)
        o_ref[...] = jnp.maximum(acc + bs_ref[...], 0.0)

    return pl.pallas_call(
        body,
        grid=grid,
        in_specs=[
            pl.BlockSpec((BLK, D_IN), lambda i: (i, 0)),
            pl.BlockSpec((NC, BLK, D_IN), lambda i: (0, i, 0)),
            pl.BlockSpec((NW, BLK), lambda i: (0, i)),
            pl.BlockSpec((D_IN, D_OUT), lambda i: (0, 0)),
            pl.BlockSpec((D_IN, D_OUT), lambda i: (0, 0)),
            pl.BlockSpec((1, D_OUT), lambda i: (0, 0)),
            pl.BlockSpec((1, D_OUT), lambda i: (0, 0)),
        ],
        out_specs=pl.BlockSpec((BLK, D_OUT), lambda i: (i, 0)),
        out_shape=jax.ShapeDtypeStruct((NPAD, D_OUT), jnp.float32),
    )(x_pad, feat, wflat, ws_t, wn_t, b_s, b_n)


def kernel(x, edge_index, time_diffs, W_S, b_S, W_N, b_N):
    n_edges = edge_index.shape[1]
    nchunks = n_edges // CH
    row = edge_index[0].astype(jnp.int32).reshape(nchunks, 2, HALF)
    col = edge_index[1].astype(jnp.int32).reshape(nchunks, 2, HALF)
    tdb = (time_diffs.astype(jnp.float32) * 16777216.0).astype(
        jnp.int32).reshape(nchunks, 2, HALF)
    pk = jnp.concatenate([row, col, tdb], axis=1)

    feat_p, wsum_p = _sc_aggregate(pk, x, nchunks)

    wflat = wsum_p.reshape(NW, NPAD)
    x_pad = jnp.pad(x, ((0, NPAD - N_NODES), (0, 0)))

    out = _tc_final(x_pad, feat_p, wflat, W_S.T, W_N.T, b_S[None, :], b_N[None, :])
    return out[:N_NODES]


# speedup vs baseline: 8.8810x; 1.0157x over previous
import functools

import jax
import jax.numpy as jnp
from jax import lax
from jax.experimental import pallas as pl
from jax.experimental.pallas import tpu as pltpu
from jax.experimental.pallas import tpu_sc as plsc

N_NODES = 10000
D_IN = 128
D_OUT = 128
NC = 2
NS = 16
NW = NC * NS
CH = 128
HALF = CH // 2
NACC = 10112
RPT = NACC // NS
WROWS = 80
NPAD = 10240
BLK = 1024


def _lane_splat(v, l):
    return lax.gather(
        v, jnp.full((16, 1), l, jnp.int32),
        lax.GatherDimensionNumbers(offset_dims=(), collapsed_slice_dims=(0,),
                                   start_index_map=(0,)),
        (1,), mode=lax.GatherScatterMode.PROMISE_IN_BOUNDS)


def _sc_aggregate(pk, x, nchunks):
    mesh = plsc.VectorSubcoreMesh(core_axis_name="c", subcore_axis_name="s")

    @functools.partial(
        pl.kernel,
        mesh=mesh,
        out_type=(
            jax.ShapeDtypeStruct((NC, NPAD, D_IN), jnp.float32),
            jax.ShapeDtypeStruct((NC, NS, WROWS, 128), jnp.float32),
        ),
        scratch_types=[
            pltpu.VMEM((3, 6, HALF), jnp.int32),
            pltpu.VMEM((CH, D_IN), jnp.float32),
            pltpu.VMEM((WROWS, 128), jnp.float32),
            pltpu.VMEM((HALF, D_IN), jnp.float32),
            pltpu.VMEM_SHARED((NACC, D_IN), jnp.float32),
            pltpu.SemaphoreType.DMA,
            pltpu.SemaphoreType.DMA,
            pltpu.SemaphoreType.DMA,
            pltpu.SemaphoreType.DMA,
            pltpu.SemaphoreType.DMA,
        ],
    )
    def k(pk_hbm, x_hbm, feat_hbm, wsum_hbm,
          pkv, gv, wloc, hc, acc, sga, sgb, ssa, ssb, si):
        cid = lax.axis_index("c")
        sid = lax.axis_index("s")
        wid = sid * NC + cid
        zero16 = jnp.zeros((16,), jnp.float32)
        lane = lax.iota(jnp.int32, 16)

        def _zw(r, carry):
            for q in range(128 // 16):
                wloc[r, pl.ds(16 * q, 16)] = zero16
            return carry
        lax.fori_loop(0, WROWS, _zw, 0)

        def _zrow(r, carry):
            for q in range(D_IN // 16):
                gv[r, pl.ds(16 * q, 16)] = zero16
            return carry
        lax.fori_loop(0, CH, _zrow, 0)
        base = sid * RPT
        for i in range(4):
            pltpu.sync_copy(gv, acc.at[pl.ds(base + i * CH, CH)])
        pltpu.sync_copy(gv.at[pl.ds(0, RPT - 4 * CH)],
                        acc.at[pl.ds(base + 4 * CH, RPT - 4 * CH)])
        plsc.subcore_barrier()

        n_my = (nchunks - wid + NW - 1) // NW

        def feat_half(ref, m, h, base):
            def grp(j, carry2):
                tdi = pkv[m, 4 + h, pl.ds(j * 16, 16)]
                w16 = jnp.exp(tdi.astype(jnp.float32) * (-1.0 / 16777216.0))
                for l in range(16):
                    wspl = _lane_splat(w16, l)
                    e = base + j * 16 + l
                    for q in range(D_IN // 16):
                        ref[e, pl.ds(16 * q, 16)] = ref[e, pl.ds(16 * q, 16)] * wspl
                return carry2
            lax.fori_loop(0, HALF // 16, grp, 0)

        def wsum_chunk(m):
            def grp(j8, carry2):
                h = j8 // (HALF // 16)
                j = j8 % (HALF // 16)
                tdi = pkv[m, 4 + h, pl.ds(j * 16, 16)]
                w16 = jnp.exp(tdi.astype(jnp.float32) * (-1.0 / 16777216.0))
                r16 = pkv[m, h, pl.ds(j * 16, 16)]
                m16 = r16 & 15
                for l in range(16):
                    mspl = _lane_splat(m16, l)
                    hot = jnp.where(lane == mspl, _lane_splat(w16, l), zero16)
                    ue = r16[l]
                    rw = lax.shift_right_logical(ue, 7)
                    c0 = pl.multiple_of((lax.shift_right_logical(ue, 4) & 7) * 16, 16)
                    wloc[rw, pl.ds(c0, 16)] = wloc[rw, pl.ds(c0, 16)] + hot
                return carry2
            lax.fori_loop(0, CH // 16, grp, 0)

        def _drain(sem, rows, nbytes_ref_rows):
            pltpu.make_async_copy(
                feat_hbm.at[0, pl.ds(0, rows)], gv.at[pl.ds(0, rows)], sem
            ).wait()

        pltpu.sync_copy(pk_hbm.at[wid], pkv.at[0])
        pltpu.async_copy(x_hbm.at[pkv.at[0, 2]], hc, sga)
        pltpu.async_copy(pk_hbm.at[wid + NW], pkv.at[1], si)

        def body(t, carry):
            par = t & 1
            m = lax.rem(t, 3)
            m1 = lax.rem(t + 1, 3)
            c = wid + t * NW

            @pl.when(t > 0)
            def _():
                _drain(ssb, HALF, None)
            gb = pltpu.async_copy(x_hbm.at[pkv.at[m, 3]], gv.at[pl.ds(HALF, HALF)], sgb)

            @pl.when(t > 0)
            def _():
                wsum_chunk(lax.rem(t + 2, 3))

            pltpu.make_async_copy(
                feat_hbm.at[0, pl.ds(0, HALF)], hc, sga).wait()

            @pl.when(par == 0)
            def _():
                feat_half(hc, m, 0, 0)
                pltpu.async_copy(hc, acc.at[pkv.at[m, 0]], ssa, add=True)

            @pl.when(par == 1)
            def _():
                feat_half(gv, m, 0, 0)
                pltpu.async_copy(gv.at[pl.ds(0, HALF)], acc.at[pkv.at[m, 0]],
                                 ssa, add=True)

            gb.wait()
            feat_half(gv, m, 1, HALF)
            pltpu.async_copy(gv.at[pl.ds(HALF, HALF)], acc.at[pkv.at[m, 1]],
                             ssb, add=True)

            @pl.when(t + 1 < n_my)
            def _():
                pltpu.make_async_copy(pk_hbm.at[0], pkv.at[m1], si).wait()

                @pl.when(t > 0)
                def _():
                    _drain(ssa, HALF, None)

                @pl.when(par == 0)
                def _():
                    pltpu.async_copy(x_hbm.at[pkv.at[m1, 2]], gv.at[pl.ds(0, HALF)], sga)

                @pl.when(par == 1)
                def _():
                    pltpu.async_copy(x_hbm.at[pkv.at[m1, 2]], hc, sga)

                @pl.when(t + 2 < n_my)
                def _():
                    pltpu.async_copy(pk_hbm.at[c + 2 * NW], pkv.at[lax.rem(t + 2, 3)], si)
            return carry
        lax.fori_loop(0, n_my, body, 0)

        wsum_chunk(lax.rem(n_my - 1, 3))
        _drain(ssa, HALF, None)
        _drain(ssa, HALF, None)
        _drain(ssb, HALF, None)
        pltpu.sync_copy(wloc, wsum_hbm.at[cid, sid])
        plsc.subcore_barrier()
        pltpu.sync_copy(acc.at[pl.ds(base, RPT)], feat_hbm.at[cid, pl.ds(base, RPT)])

    return k(pk, x)


def _tc_final(x_pad, feat, wflat, ws_t, wn_t, b_s, b_n):
    grid = (NPAD // BLK,)

    def body(x_ref, pf_ref, wf_ref, ws_ref, wn_ref, bs_ref, bn_ref, o_ref):
        agg = pf_ref[0] + pf_ref[1]
        acc = jnp.dot(x_ref[...], ws_ref[...], preferred_element_type=jnp.float32)
        acc += jnp.dot(agg, wn_ref[...], preferred_element_type=jnp.float32)
        bn_b = jnp.broadcast_to(bn_ref[...], (NW, D_OUT))
        acc += lax.dot_general(wf_ref[...], bn_b, (((0,), (0,)), ((), ())),
                               preferred_element_type=jnp.float32)
        o_ref[...] = jnp.maximum(acc + bs_ref[...], 0.0)

    return pl.pallas_call(
        body,
        grid=grid,
        in_specs=[
            pl.BlockSpec((BLK, D_IN), lambda i: (i, 0)),
            pl.BlockSpec((NC, BLK, D_IN), lambda i: (0, i, 0)),
            pl.BlockSpec((NW, BLK), lambda i: (0, i)),
            pl.BlockSpec((D_IN, D_OUT), lambda i: (0, 0)),
            pl.BlockSpec((D_IN, D_OUT), lambda i: (0, 0)),
            pl.BlockSpec((1, D_OUT), lambda i: (0, 0)),
            pl.BlockSpec((1, D_OUT), lambda i: (0, 0)),
        ],
        out_specs=pl.BlockSpec((BLK, D_OUT), lambda i: (i, 0)),
        out_shape=jax.ShapeDtypeStruct((NPAD, D_OUT), jnp.float32),
    )(x_pad, feat, wflat, ws_t, wn_t, b_s, b_n)


def kernel(x, edge_index, time_diffs, W_S, b_S, W_N, b_N):
    n_edges = edge_index.shape[1]
    nchunks = n_edges // CH
    row = edge_index[0].astype(jnp.int32).reshape(nchunks, 2, HALF)
    col = edge_index[1].astype(jnp.int32).reshape(nchunks, 2, HALF)
    tdb = (time_diffs.astype(jnp.float32) * 16777216.0).astype(
        jnp.int32).reshape(nchunks, 2, HALF)
    pk = jnp.concatenate([row, col, tdb], axis=1)

    feat_p, wsum_p = _sc_aggregate(pk, x, nchunks)

    wflat = wsum_p.reshape(NW, NPAD)
    x_pad = jnp.pad(x, ((0, NPAD - N_NODES), (0, 0)))

    out = _tc_final(x_pad, feat_p, wflat, W_S.T, W_N.T, b_S[None, :], b_N[None, :])
    return out[:N_NODES]
